# Initial kernel scaffold; baseline (speedup 1.0000x reference)
import jax
import jax.numpy as jnp
from jax.experimental import pallas as pl


def kernel(words, W):
    raise NotImplementedError("write your pallas kernel here")


# speedup vs baseline: 4.1389x; 4.1389x over previous
import functools

import jax
import jax.numpy as jnp
from jax import lax
from jax.experimental import pallas as pl
from jax.experimental.pallas import tpu as pltpu
from jax.experimental.pallas import tpu_sc as plsc

VOCAB = 100000
EMBED_DIM = 64
BATCH = 4096
HIST = 200

NC = 2
NS = 16
NW = NC * NS

TOTAL = BATCH * HIST
PER_W = TOTAL // NW
CHUNK = 1024
NCHUNK = PER_W // CHUNK

_mesh = plsc.VectorSubcoreMesh(
    core_axis_name="c", subcore_axis_name="s", num_cores=NC, num_subcores=NS
)


@functools.partial(
    pl.kernel,
    out_type=jax.ShapeDtypeStruct((TOTAL, EMBED_DIM), jnp.float32),
    mesh=_mesh,
    scratch_types=[
        pltpu.VMEM((CHUNK,), jnp.int32),
        pltpu.VMEM((CHUNK, EMBED_DIM), jnp.float32),
        pltpu.SemaphoreType.DMA,
    ],
    compiler_params=pltpu.CompilerParams(use_tc_tiling_on_sc=False),
)
def _gather_kernel(idx_hbm, table_hbm, out_hbm, idx_v, rows_v, sem):
    wid = lax.axis_index("s") * NC + lax.axis_index("c")
    wbase = wid * PER_W

    def step(i, carry):
        base = wbase + i * CHUNK
        pltpu.sync_copy(idx_hbm.at[pl.ds(base, CHUNK)], idx_v)
        pltpu.async_copy(table_hbm.at[idx_v], rows_v, sem).wait()
        pltpu.sync_copy(rows_v, out_hbm.at[pl.ds(base, CHUNK)])
        return carry

    lax.fori_loop(0, NCHUNK, step, 0)


def kernel(words, W):
    idx = words.reshape(TOTAL).astype(jnp.int32)
    out = _gather_kernel(idx, W)
    return out.reshape(BATCH, HIST, EMBED_DIM)


# speedup vs baseline: 4.2481x; 1.0264x over previous
import functools

import jax
import jax.numpy as jnp
from jax import lax
from jax.experimental import pallas as pl
from jax.experimental.pallas import tpu as pltpu
from jax.experimental.pallas import tpu_sc as plsc

VOCAB = 100000
EMBED_DIM = 64
BATCH = 4096
HIST = 200

NC = 2
NS = 16
NW = NC * NS

TOTAL = BATCH * HIST
PER_W = TOTAL // NW
CHUNK = 800
NCHUNK = PER_W // CHUNK

_mesh = plsc.VectorSubcoreMesh(
    core_axis_name="c", subcore_axis_name="s", num_cores=NC, num_subcores=NS
)


@functools.partial(
    pl.kernel,
    out_type=jax.ShapeDtypeStruct((TOTAL, EMBED_DIM), jnp.float32),
    mesh=_mesh,
    scratch_types=[
        pltpu.VMEM((PER_W,), jnp.int32),
        pltpu.VMEM((CHUNK, EMBED_DIM), jnp.float32),
        pltpu.VMEM((CHUNK, EMBED_DIM), jnp.float32),
        pltpu.SemaphoreType.DMA,
        pltpu.SemaphoreType.DMA,
        pltpu.SemaphoreType.DMA,
        pltpu.SemaphoreType.DMA,
    ],
    compiler_params=pltpu.CompilerParams(use_tc_tiling_on_sc=False),
)
def _gather_kernel(idx_hbm, table_hbm, out_hbm, idx_v, rows0, rows1,
                   gsem0, gsem1, ssem0, ssem1):
    wid = lax.axis_index("s") * NC + lax.axis_index("c")
    wbase = wid * PER_W
    rows = (rows0, rows1)
    gsem = (gsem0, gsem1)
    ssem = (ssem0, ssem1)

    def start_gather(g, b):
        pltpu.async_copy(
            table_hbm.at[idx_v.at[pl.ds(g * CHUNK, CHUNK)]], rows[b], gsem[b])

    def wait_gather(b):
        pltpu.make_async_copy(
            table_hbm.at[idx_v.at[pl.ds(0, CHUNK)]], rows[b], gsem[b]).wait()

    def start_scatter(g, b):
        pltpu.async_copy(
            rows[b], out_hbm.at[pl.ds(wbase + g * CHUNK, CHUNK)], ssem[b])

    def wait_scatter(b):
        pltpu.make_async_copy(
            rows[b], out_hbm.at[pl.ds(wbase, CHUNK)], ssem[b]).wait()

    pltpu.sync_copy(idx_hbm.at[pl.ds(wbase, PER_W)], idx_v)

    start_gather(0, 0)
    wait_gather(0)
    start_gather(1, 1)
    start_scatter(0, 0)

    def step(h, carry):
        for t in range(2):
            g = h * 2 + 1 + t
            b = (1 + t) % 2
            bn = t % 2
            wait_gather(b)
            wait_scatter(bn)
            start_gather(g + 1, bn)
            start_scatter(g, b)
        return carry

    lax.fori_loop(0, (NCHUNK - 2) // 2, step, 0)

    g = NCHUNK - 1
    b = g % 2
    wait_gather(b)
    start_scatter(g, b)
    wait_scatter((g + 1) % 2)
    wait_scatter(b)


def kernel(words, W):
    idx = words.reshape(TOTAL).astype(jnp.int32)
    out = _gather_kernel(idx, W)
    return out.reshape(BATCH, HIST, EMBED_DIM)


# speedup vs baseline: 4.2585x; 1.0025x over previous
import functools

import jax
import jax.numpy as jnp
from jax import lax
from jax.experimental import pallas as pl
from jax.experimental.pallas import tpu as pltpu
from jax.experimental.pallas import tpu_sc as plsc

VOCAB = 100000
EMBED_DIM = 64
BATCH = 4096
HIST = 200

NC = 2
NS = 16
NW = NC * NS

TOTAL = BATCH * HIST
PER_W = TOTAL // NW
CHUNK = 400
NCHUNK = PER_W // CHUNK
NBUF = 4
K = 2

assert PER_W % CHUNK == 0 and NCHUNK % NBUF == 0 and 0 < K < NBUF

_mesh = plsc.VectorSubcoreMesh(
    core_axis_name="c", subcore_axis_name="s", num_cores=NC, num_subcores=NS
)


@functools.partial(
    pl.kernel,
    out_type=jax.ShapeDtypeStruct((TOTAL, EMBED_DIM), jnp.float32),
    mesh=_mesh,
    scratch_types=[
        pltpu.VMEM((PER_W,), jnp.int32),
        [pltpu.VMEM((CHUNK, EMBED_DIM), jnp.float32)] * NBUF,
        [pltpu.SemaphoreType.DMA] * NBUF,
        [pltpu.SemaphoreType.DMA] * NBUF,
    ],
    compiler_params=pltpu.CompilerParams(use_tc_tiling_on_sc=False),
)
def _gather_kernel(idx_hbm, table_hbm, out_hbm, idx_v, rows, gsem, ssem):
    wid = lax.axis_index("s") * NC + lax.axis_index("c")
    wbase = wid * PER_W

    def start_gather(g, b):
        pltpu.async_copy(
            table_hbm.at[idx_v.at[pl.ds(g * CHUNK, CHUNK)]], rows[b], gsem[b])

    def wait_gather(b):
        pltpu.make_async_copy(
            table_hbm.at[idx_v.at[pl.ds(0, CHUNK)]], rows[b], gsem[b]).wait()

    def start_scatter(g, b):
        pltpu.async_copy(
            rows[b], out_hbm.at[pl.ds(wbase + g * CHUNK, CHUNK)], ssem[b])

    def wait_scatter(b):
        pltpu.make_async_copy(
            rows[b], out_hbm.at[pl.ds(wbase, CHUNK)], ssem[b]).wait()

    pltpu.sync_copy(idx_hbm.at[pl.ds(wbase, PER_W)], idx_v)

    for j in range(K):
        start_gather(j, j % NBUF)

    for g in range(NBUF - K):
        b = g % NBUF
        wait_gather(b)
        start_scatter(g, b)
        start_gather(g + K, (g + K) % NBUF)

    def step(h, carry):
        for t in range(NBUF):
            g = (NBUF - K) + h * NBUF + t
            b = (NBUF - K + t) % NBUF
            bj = (b + K) % NBUF
            wait_gather(b)
            start_scatter(g, b)
            wait_scatter(bj)
            start_gather(g + K, bj)
        return carry

    lax.fori_loop(0, (NCHUNK - NBUF) // NBUF, step, 0)

    for g in range(NCHUNK - K, NCHUNK):
        b = g % NBUF
        wait_gather(b)
        start_scatter(g, b)
    for b in range(NBUF):
        wait_scatter(b)


def kernel(words, W):
    idx = words.reshape(TOTAL).astype(jnp.int32)
    out = _gather_kernel(idx, W)
    return out.reshape(BATCH, HIST, EMBED_DIM)


# speedup vs baseline: 4.2651x; 1.0015x over previous
import functools

import jax
import jax.numpy as jnp
from jax import lax
from jax.experimental import pallas as pl
from jax.experimental.pallas import tpu as pltpu
from jax.experimental.pallas import tpu_sc as plsc

VOCAB = 100000
EMBED_DIM = 64
BATCH = 4096
HIST = 200

NC = 2
NS = 16
NW = NC * NS

TOTAL = BATCH * HIST
PER_W = TOTAL // NW
CHUNK = 200
NCHUNK = PER_W // CHUNK
NBUF = 8
K = 4

assert PER_W % CHUNK == 0 and NCHUNK % NBUF == 0 and 0 < K < NBUF

_mesh = plsc.VectorSubcoreMesh(
    core_axis_name="c", subcore_axis_name="s", num_cores=NC, num_subcores=NS
)


@functools.partial(
    pl.kernel,
    out_type=jax.ShapeDtypeStruct((TOTAL, EMBED_DIM), jnp.float32),
    mesh=_mesh,
    scratch_types=[
        pltpu.VMEM((PER_W,), jnp.int32),
        [pltpu.VMEM((CHUNK, EMBED_DIM), jnp.float32)] * NBUF,
        [pltpu.SemaphoreType.DMA] * NBUF,
        [pltpu.SemaphoreType.DMA] * NBUF,
    ],
    compiler_params=pltpu.CompilerParams(use_tc_tiling_on_sc=False),
)
def _gather_kernel(idx_hbm, table_hbm, out_hbm, idx_v, rows, gsem, ssem):
    wid = lax.axis_index("s") * NC + lax.axis_index("c")
    wbase = wid * PER_W

    def start_gather(g, b):
        pltpu.async_copy(
            table_hbm.at[idx_v.at[pl.ds(g * CHUNK, CHUNK)]], rows[b], gsem[b])

    def wait_gather(b):
        pltpu.make_async_copy(
            table_hbm.at[idx_v.at[pl.ds(0, CHUNK)]], rows[b], gsem[b]).wait()

    def start_scatter(g, b):
        pltpu.async_copy(
            rows[b], out_hbm.at[pl.ds(wbase + g * CHUNK, CHUNK)], ssem[b])

    def wait_scatter(b):
        pltpu.make_async_copy(
            rows[b], out_hbm.at[pl.ds(wbase, CHUNK)], ssem[b]).wait()

    pltpu.sync_copy(idx_hbm.at[pl.ds(wbase, PER_W)], idx_v)

    for j in range(K):
        start_gather(j, j % NBUF)

    for g in range(NBUF - K):
        b = g % NBUF
        wait_gather(b)
        start_scatter(g, b)
        start_gather(g + K, (g + K) % NBUF)

    def step(h, carry):
        for t in range(NBUF):
            g = (NBUF - K) + h * NBUF + t
            b = (NBUF - K + t) % NBUF
            bj = (b + K) % NBUF
            wait_gather(b)
            start_scatter(g, b)
            wait_scatter(bj)
            start_gather(g + K, bj)
        return carry

    lax.fori_loop(0, (NCHUNK - NBUF) // NBUF, step, 0)

    for g in range(NCHUNK - K, NCHUNK):
        b = g % NBUF
        wait_gather(b)
        start_scatter(g, b)
    for b in range(NBUF):
        wait_scatter(b)


def kernel(words, W):
    idx = words.reshape(TOTAL).astype(jnp.int32)
    out = _gather_kernel(idx, W)
    return out.reshape(BATCH, HIST, EMBED_DIM)
